# Initial kernel scaffold; baseline (speedup 1.0000x reference)
import jax
import jax.numpy as jnp
from jax.experimental import pallas as pl


def kernel(xyz, field, point_xyz, child_index, point_index, child_cut, activation_layer):
    raise NotImplementedError("write your pallas kernel here")


# speedup vs baseline: 352.6586x; 352.6586x over previous
import functools

import jax
import jax.numpy as jnp
from jax import lax
from jax.experimental import pallas as pl
from jax.experimental.pallas import tpu as pltpu
from jax.experimental.pallas import tpu_sc as plsc

_LAYERS = 16
_N = 65536
_F = 32
_NC, _NS, _L = 2, 16, 16
_NW = _NC * _NS
_P = _N // _NW
_C = _P // _L
_G = 128
_NG = _P // _G
_GB = 128
_Q = _P // 2
_W = 16


def _sel4(idx, v0, v1, v2, v3):
    return jnp.where(idx == 0, v0, jnp.where(idx == 1, v1, jnp.where(idx == 2, v2, v3)))


def _body(xyz3, field, ainv, packed2, out,
          xyzv, ainv_v, initp,
          b0, b1, b2, b3, p0, p1, p2, p3,
          cellv, ch0v, ch1v, cut0v, cut1v, actv,
          av, kv, gidx, childv, npack,
          rows0, rows1, rows2, rows3, outstage, sem):
    wid = lax.axis_index("s") * _NC + lax.axis_index("c")
    base = pl.multiple_of(wid * _P, 8)
    lane = jax.lax.iota(jnp.int32, _L)

    pltpu.sync_copy(xyz3.at[pl.ds(base * 3, _P * 3)], xyzv)
    pltpu.sync_copy(ainv, ainv_v)
    pltpu.sync_copy(packed2.at[pl.ds(0, 1)], initp)

    avec = ainv_v[...]
    a = [[avec[4 * r + c] for c in range(4)] for r in range(4)]
    pvec = initp[0]
    ch0s, ch1s = pvec[0], pvec[1]
    cut0s, cut1s = pvec[2], pvec[3]
    act0s = pvec[4]
    brefs = (b0, b1, b2, b3)
    prefs = (p0, p1, p2, p3)

    def init_chunk(c, carry):
        s = pl.ds(pl.multiple_of(c * _L, 8), _L)
        i3 = (c * _L + lane) * 3
        x = plsc.load_gather(xyzv, [i3])
        y = plsc.load_gather(xyzv, [i3 + 1])
        z = plsc.load_gather(xyzv, [i3 + 2])
        for r in range(4):
            brefs[r][s] = a[r][0] * x + a[r][1] * y + a[r][2] * z + a[r][3]
            prefs[r][s] = jnp.full((_L,), r, jnp.int32)
        cellv[s] = jnp.zeros((_L,), jnp.int32)
        ch0v[s] = jnp.full((_L,), ch0s, jnp.int32)
        ch1v[s] = jnp.full((_L,), ch1s, jnp.int32)
        cut0v[s] = jnp.full((_L,), cut0s, jnp.int32)
        cut1v[s] = jnp.full((_L,), cut1s, jnp.int32)
        actv[s] = jnp.full((_L,), act0s, jnp.int32)
        return carry

    lax.fori_loop(0, _C, init_chunk, 0)

    def layer(i, carry):
        def pre(c, cr):
            s = pl.ds(pl.multiple_of(c * _L, 8), _L)
            bs = (b0[s], b1[s], b2[s], b3[s])
            c0, c1 = cut0v[s], cut1v[s]
            bc0 = _sel4(c0, *bs)
            bc1 = _sel4(c1, *bs)
            ch = bc0 - bc1 > 0
            av[s] = jnp.where(ch, c0, c1)
            kv[s] = jnp.where(ch, c1, c0)
            child = jnp.where(ch, ch1v[s], ch0v[s])
            childv[s] = child
            gidx[s] = jnp.maximum(child, 0)
            return cr

        lax.fori_loop(0, _C, pre, 0)

        def fire(g, cr):
            o = pl.ds(pl.multiple_of(g * _G, 8), _G)
            pltpu.async_copy(packed2.at[gidx.at[o]], npack.at[o], sem)
            return cr

        def drain(g, cr):
            o = pl.ds(pl.multiple_of(g * _G, 8), _G)
            pltpu.make_async_copy(packed2.at[gidx.at[o]], npack.at[o], sem).wait()
            return cr

        lax.fori_loop(0, _NG, fire, 0)
        lax.fori_loop(0, _NG, drain, 0)

        def post(c, cr):
            s = pl.ds(pl.multiple_of(c * _L, 8), _L)
            chd = childv[s]
            valid = ((chd != -1) & (cellv[s] != -1)
                     & (jnp.minimum(actv[s], _LAYERS) == i))
            aa, kk = av[s], kv[s]
            ri = c * _L + lane
            nch0 = plsc.load_gather(npack, [ri, jnp.full((_L,), 0, jnp.int32)])
            nch1 = plsc.load_gather(npack, [ri, jnp.full((_L,), 1, jnp.int32)])
            ncut0 = plsc.load_gather(npack, [ri, jnp.full((_L,), 2, jnp.int32)])
            ncut1 = plsc.load_gather(npack, [ri, jnp.full((_L,), 3, jnp.int32)])
            nact = plsc.load_gather(npack, [ri, jnp.full((_L,), 4, jnp.int32)])
            vid = plsc.load_gather(npack, [ri, 5 + aa])
            bs = (b0[s], b1[s], b2[s], b3[s])
            ba = _sel4(aa, *bs)
            for r in range(4):
                isa = aa == r
                isk = kk == r
                nbr = jnp.where(isa, 2.0 * ba, jnp.where(isk, bs[r] - ba, bs[r]))
                brefs[r][s] = jnp.where(valid, nbr, bs[r])
                pr = prefs[r][s]
                prefs[r][s] = jnp.where(valid & isa, vid, pr)
            cellv[s] = jnp.where(valid, chd, cellv[s])
            ch0v[s] = jnp.where(valid, nch0, ch0v[s])
            ch1v[s] = jnp.where(valid, nch1, ch1v[s])
            cut0v[s] = jnp.where(valid, ncut0, cut0v[s])
            cut1v[s] = jnp.where(valid, ncut1, cut1v[s])
            actv[s] = jnp.where(valid, nact, actv[s])
            return cr

        lax.fori_loop(0, _C, post, 0)
        return carry

    lax.fori_loop(1, _LAYERS, layer, 0)

    rows = (rows0, rows1, rows2, rows3)

    def interp_block(blk, carry):
        off = pl.multiple_of(blk * _GB, 8)
        st0 = off - (off // _Q) * _Q
        o = pl.ds(off, _GB)
        for r in range(4):
            pltpu.async_copy(field.at[prefs[r].at[o]], rows[r], sem)
        for r in range(4):
            pltpu.make_async_copy(field.at[prefs[r].at[o]], rows[r], sem).wait()

        def sub(j, cr):
            ps = pl.ds(pl.multiple_of(off + j * _L, 8), _L)
            w = [b0[ps], b1[ps], b2[ps], b3[ps]]
            rowidx = j * _L + lane
            stpos = st0 + j * _L + lane
            for f in range(_F):
                col = jnp.full((_L,), f, jnp.int32)
                acc = w[0] * plsc.load_gather(rows0, [rowidx, col])
                acc = acc + w[1] * plsc.load_gather(rows1, [rowidx, col])
                acc = acc + w[2] * plsc.load_gather(rows2, [rowidx, col])
                acc = acc + w[3] * plsc.load_gather(rows3, [rowidx, col])
                plsc.store_scatter(outstage, [stpos, col], acc)
            return cr

        lax.fori_loop(0, _GB // _L, sub, 0)
        return carry

    nblk_half = _Q // _GB
    for half in range(2):
        lax.fori_loop(half * nblk_half, (half + 1) * nblk_half, interp_block, 0)
        pltpu.sync_copy(outstage, out.at[pl.ds(base + half * _Q, _Q)])


_vmem = pltpu.VMEM
_scratch = [
    _vmem((_P * 3,), jnp.float32),
    _vmem((16,), jnp.float32), _vmem((1, _W), jnp.int32),
    _vmem((_P,), jnp.float32), _vmem((_P,), jnp.float32),
    _vmem((_P,), jnp.float32), _vmem((_P,), jnp.float32),
    _vmem((_P,), jnp.int32), _vmem((_P,), jnp.int32),
    _vmem((_P,), jnp.int32), _vmem((_P,), jnp.int32),
    _vmem((_P,), jnp.int32), _vmem((_P,), jnp.int32),
    _vmem((_P,), jnp.int32), _vmem((_P,), jnp.int32),
    _vmem((_P,), jnp.int32), _vmem((_P,), jnp.int32),
    _vmem((_P,), jnp.int32), _vmem((_P,), jnp.int32),
    _vmem((_P,), jnp.int32), _vmem((_P,), jnp.int32),
    _vmem((_P, _W), jnp.int32),
    _vmem((_GB, _F), jnp.float32), _vmem((_GB, _F), jnp.float32),
    _vmem((_GB, _F), jnp.float32), _vmem((_GB, _F), jnp.float32),
    _vmem((_Q, _F), jnp.float32),
    pltpu.SemaphoreType.DMA,
]

_descend = functools.partial(
    pl.kernel,
    out_type=jax.ShapeDtypeStruct((_N, _F), jnp.float32),
    mesh=plsc.VectorSubcoreMesh(core_axis_name="c", subcore_axis_name="s"),
    compiler_params=pltpu.CompilerParams(
        needs_layout_passes=False, use_tc_tiling_on_sc=False),
    scratch_types=_scratch,
)(_body)


def kernel(xyz, field, point_xyz, child_index, point_index, child_cut, activation_layer):
    m = jnp.concatenate(
        [point_xyz[:4], jnp.ones((4, 1), point_xyz.dtype)], axis=1).T
    ainv = jnp.linalg.inv(m).astype(jnp.float32).reshape(16)
    ncell = child_index.shape[0]
    packed = jnp.concatenate(
        [child_index.astype(jnp.int32),
         child_cut.astype(jnp.int32),
         activation_layer.astype(jnp.int32)[:, None],
         point_index.astype(jnp.int32),
         jnp.zeros((ncell, _W - 9), jnp.int32)],
        axis=1)
    return _descend(xyz.reshape(-1), field, ainv, packed)


# speedup vs baseline: 353.9301x; 1.0036x over previous
import functools

import jax
import jax.numpy as jnp
from jax import lax
from jax.experimental import pallas as pl
from jax.experimental.pallas import tpu as pltpu
from jax.experimental.pallas import tpu_sc as plsc

_LAYERS = 16
_N = 65536
_F = 32
_NC, _NS, _L = 2, 16, 16
_NW = _NC * _NS
_P = _N // _NW
_C = _P // _L
_G = 32
_NG = _P // _G
_GB = 128
_Q = _P // 2
_W = 16


def _sel4(idx, v0, v1, v2, v3):
    return jnp.where(idx == 0, v0, jnp.where(idx == 1, v1, jnp.where(idx == 2, v2, v3)))


def _body(xyz3, field, ainv, packed2, out,
          xyzv, ainv_v, initp,
          b0, b1, b2, b3, p0, p1, p2, p3,
          cellv, ch0v, ch1v, cut0v, cut1v, actv,
          av, kv, gidx, childv, npack,
          rows0, rows1, rows2, rows3, outstage, sem):
    wid = lax.axis_index("s") * _NC + lax.axis_index("c")
    base = pl.multiple_of(wid * _P, 8)
    lane = jax.lax.iota(jnp.int32, _L)

    pltpu.sync_copy(xyz3.at[pl.ds(base * 3, _P * 3)], xyzv)
    pltpu.sync_copy(ainv, ainv_v)
    pltpu.sync_copy(packed2.at[pl.ds(0, 1)], initp)

    avec = ainv_v[...]
    a = [[avec[4 * r + c] for c in range(4)] for r in range(4)]
    pvec = initp[0]
    ch0s, ch1s = pvec[0], pvec[1]
    cut0s, cut1s = pvec[2], pvec[3]
    act0s = pvec[4]
    brefs = (b0, b1, b2, b3)
    prefs = (p0, p1, p2, p3)

    def init_chunk(c, carry):
        s = pl.ds(pl.multiple_of(c * _L, 8), _L)
        i3 = (c * _L + lane) * 3
        x = plsc.load_gather(xyzv, [i3])
        y = plsc.load_gather(xyzv, [i3 + 1])
        z = plsc.load_gather(xyzv, [i3 + 2])
        for r in range(4):
            brefs[r][s] = a[r][0] * x + a[r][1] * y + a[r][2] * z + a[r][3]
            prefs[r][s] = jnp.full((_L,), r, jnp.int32)
        cellv[s] = jnp.zeros((_L,), jnp.int32)
        ch0v[s] = jnp.full((_L,), ch0s, jnp.int32)
        ch1v[s] = jnp.full((_L,), ch1s, jnp.int32)
        cut0v[s] = jnp.full((_L,), cut0s, jnp.int32)
        cut1v[s] = jnp.full((_L,), cut1s, jnp.int32)
        actv[s] = jnp.full((_L,), act0s, jnp.int32)
        return carry

    lax.fori_loop(0, _C, init_chunk, 0)

    def layer(i, carry):
        def pre(c, cr):
            s = pl.ds(pl.multiple_of(c * _L, 8), _L)
            bs = (b0[s], b1[s], b2[s], b3[s])
            c0, c1 = cut0v[s], cut1v[s]
            bc0 = _sel4(c0, *bs)
            bc1 = _sel4(c1, *bs)
            ch = bc0 - bc1 > 0
            av[s] = jnp.where(ch, c0, c1)
            kv[s] = jnp.where(ch, c1, c0)
            child = jnp.where(ch, ch1v[s], ch0v[s])
            childv[s] = child
            gidx[s] = jnp.maximum(child, 0)
            return cr

        lax.fori_loop(0, _C, pre, 0)

        def fire(g, cr):
            o = pl.ds(pl.multiple_of(g * _G, 8), _G)
            pltpu.async_copy(packed2.at[gidx.at[o]], npack.at[o], sem)
            return cr

        def drain(g, cr):
            o = pl.ds(pl.multiple_of(g * _G, 8), _G)
            pltpu.make_async_copy(packed2.at[gidx.at[o]], npack.at[o], sem).wait()
            return cr

        lax.fori_loop(0, _NG, fire, 0)
        lax.fori_loop(0, _NG, drain, 0)

        def post(c, cr):
            s = pl.ds(pl.multiple_of(c * _L, 8), _L)
            chd = childv[s]
            valid = ((chd != -1) & (cellv[s] != -1)
                     & (jnp.minimum(actv[s], _LAYERS) == i))
            aa, kk = av[s], kv[s]
            ri = c * _L + lane
            nch0 = plsc.load_gather(npack, [ri, jnp.full((_L,), 0, jnp.int32)])
            nch1 = plsc.load_gather(npack, [ri, jnp.full((_L,), 1, jnp.int32)])
            ncut0 = plsc.load_gather(npack, [ri, jnp.full((_L,), 2, jnp.int32)])
            ncut1 = plsc.load_gather(npack, [ri, jnp.full((_L,), 3, jnp.int32)])
            nact = plsc.load_gather(npack, [ri, jnp.full((_L,), 4, jnp.int32)])
            vid = plsc.load_gather(npack, [ri, 5 + aa])
            bs = (b0[s], b1[s], b2[s], b3[s])
            ba = _sel4(aa, *bs)
            for r in range(4):
                isa = aa == r
                isk = kk == r
                nbr = jnp.where(isa, 2.0 * ba, jnp.where(isk, bs[r] - ba, bs[r]))
                brefs[r][s] = jnp.where(valid, nbr, bs[r])
                pr = prefs[r][s]
                prefs[r][s] = jnp.where(valid & isa, vid, pr)
            cellv[s] = jnp.where(valid, chd, cellv[s])
            ch0v[s] = jnp.where(valid, nch0, ch0v[s])
            ch1v[s] = jnp.where(valid, nch1, ch1v[s])
            cut0v[s] = jnp.where(valid, ncut0, cut0v[s])
            cut1v[s] = jnp.where(valid, ncut1, cut1v[s])
            actv[s] = jnp.where(valid, nact, actv[s])
            return cr

        lax.fori_loop(0, _C, post, 0)
        return carry

    lax.fori_loop(1, _LAYERS, layer, 0)

    rows = (rows0, rows1, rows2, rows3)

    def interp_block(blk, carry):
        off = pl.multiple_of(blk * _GB, 8)
        st0 = off - (off // _Q) * _Q
        for r in range(4):
            for q in range(4):
                oq = pl.ds(pl.multiple_of(off + q * 32, 8), 32)
                dq = pl.ds(pl.multiple_of(q * 32, 8), 32)
                pltpu.async_copy(field.at[prefs[r].at[oq]], rows[r].at[dq], sem)
        for r in range(4):
            for q in range(4):
                oq = pl.ds(pl.multiple_of(off + q * 32, 8), 32)
                dq = pl.ds(pl.multiple_of(q * 32, 8), 32)
                pltpu.make_async_copy(field.at[prefs[r].at[oq]], rows[r].at[dq], sem).wait()

        def sub(j, cr):
            ps = pl.ds(pl.multiple_of(off + j * _L, 8), _L)
            w = [b0[ps], b1[ps], b2[ps], b3[ps]]
            rowidx = j * _L + lane
            stpos = st0 + j * _L + lane
            for f in range(_F):
                col = jnp.full((_L,), f, jnp.int32)
                acc = w[0] * plsc.load_gather(rows0, [rowidx, col])
                acc = acc + w[1] * plsc.load_gather(rows1, [rowidx, col])
                acc = acc + w[2] * plsc.load_gather(rows2, [rowidx, col])
                acc = acc + w[3] * plsc.load_gather(rows3, [rowidx, col])
                plsc.store_scatter(outstage, [stpos, col], acc)
            return cr

        lax.fori_loop(0, _GB // _L, sub, 0)
        return carry

    nblk_half = _Q // _GB
    for half in range(2):
        lax.fori_loop(half * nblk_half, (half + 1) * nblk_half, interp_block, 0)
        pltpu.sync_copy(outstage, out.at[pl.ds(base + half * _Q, _Q)])


_vmem = pltpu.VMEM
_scratch = [
    _vmem((_P * 3,), jnp.float32),
    _vmem((16,), jnp.float32), _vmem((1, _W), jnp.int32),
    _vmem((_P,), jnp.float32), _vmem((_P,), jnp.float32),
    _vmem((_P,), jnp.float32), _vmem((_P,), jnp.float32),
    _vmem((_P,), jnp.int32), _vmem((_P,), jnp.int32),
    _vmem((_P,), jnp.int32), _vmem((_P,), jnp.int32),
    _vmem((_P,), jnp.int32), _vmem((_P,), jnp.int32),
    _vmem((_P,), jnp.int32), _vmem((_P,), jnp.int32),
    _vmem((_P,), jnp.int32), _vmem((_P,), jnp.int32),
    _vmem((_P,), jnp.int32), _vmem((_P,), jnp.int32),
    _vmem((_P,), jnp.int32), _vmem((_P,), jnp.int32),
    _vmem((_P, _W), jnp.int32),
    _vmem((_GB, _F), jnp.float32), _vmem((_GB, _F), jnp.float32),
    _vmem((_GB, _F), jnp.float32), _vmem((_GB, _F), jnp.float32),
    _vmem((_Q, _F), jnp.float32),
    pltpu.SemaphoreType.DMA,
]

_descend = functools.partial(
    pl.kernel,
    out_type=jax.ShapeDtypeStruct((_N, _F), jnp.float32),
    mesh=plsc.VectorSubcoreMesh(core_axis_name="c", subcore_axis_name="s"),
    compiler_params=pltpu.CompilerParams(
        needs_layout_passes=False, use_tc_tiling_on_sc=False),
    scratch_types=_scratch,
)(_body)


def kernel(xyz, field, point_xyz, child_index, point_index, child_cut, activation_layer):
    m = jnp.concatenate(
        [point_xyz[:4], jnp.ones((4, 1), point_xyz.dtype)], axis=1).T
    ainv = jnp.linalg.inv(m).astype(jnp.float32).reshape(16)
    ncell = child_index.shape[0]
    packed = jnp.concatenate(
        [child_index.astype(jnp.int32),
         child_cut.astype(jnp.int32),
         activation_layer.astype(jnp.int32)[:, None],
         point_index.astype(jnp.int32),
         jnp.zeros((ncell, _W - 9), jnp.int32)],
        axis=1)
    return _descend(xyz.reshape(-1), field, ainv, packed)


# speedup vs baseline: 606.3078x; 1.7131x over previous
import functools

import jax
import jax.numpy as jnp
from jax import lax
from jax.experimental import pallas as pl
from jax.experimental.pallas import tpu as pltpu
from jax.experimental.pallas import tpu_sc as plsc

_LAYERS = 16
_N = 65536
_F = 32
_NC, _NS, _L = 2, 16, 16
_NW = _NC * _NS
_P = _N // _NW
_C = _P // _L
_G = 32
_NG = _P // _G
_GB = 128
_Q = _P // 4
_TV = 1024
_W = 16


def _sel4(idx, v0, v1, v2, v3):
    return jnp.where(idx == 0, v0, jnp.where(idx == 1, v1, jnp.where(idx == 2, v2, v3)))


def _body(xyz3, field, ainv, packed2, out,
          vtab, xyzv, ainv_v, initp,
          b0, b1, b2, b3, p0, p1, p2, p3,
          cellv, ch0v, ch1v, cut0v, cut1v, actv,
          av, kv, gidx, childv, npack,
          rows0, rows1, rows2, rows3, outstage, sem):
    wid = lax.axis_index("s") * _NC + lax.axis_index("c")
    base = pl.multiple_of(wid * _P, 8)
    lane = jax.lax.iota(jnp.int32, _L)

    pltpu.sync_copy(xyz3.at[pl.ds(base * 3, _P * 3)], xyzv)
    pltpu.sync_copy(ainv, ainv_v)
    pltpu.sync_copy(packed2.at[pl.ds(0, 1)], initp)
    pltpu.sync_copy(packed2.at[pl.ds(0, _TV)], vtab)

    avec = ainv_v[...]
    a = [[avec[4 * r + c] for c in range(4)] for r in range(4)]
    pvec = initp[0]
    ch0s, ch1s = pvec[0], pvec[1]
    cut0s, cut1s = pvec[2], pvec[3]
    act0s = pvec[4]
    brefs = (b0, b1, b2, b3)
    prefs = (p0, p1, p2, p3)

    def init_chunk(c, carry):
        s = pl.ds(pl.multiple_of(c * _L, 8), _L)
        i3 = (c * _L + lane) * 3
        x = plsc.load_gather(xyzv, [i3])
        y = plsc.load_gather(xyzv, [i3 + 1])
        z = plsc.load_gather(xyzv, [i3 + 2])
        for r in range(4):
            brefs[r][s] = a[r][0] * x + a[r][1] * y + a[r][2] * z + a[r][3]
            prefs[r][s] = jnp.full((_L,), r, jnp.int32)
        cellv[s] = jnp.zeros((_L,), jnp.int32)
        ch0v[s] = jnp.full((_L,), ch0s, jnp.int32)
        ch1v[s] = jnp.full((_L,), ch1s, jnp.int32)
        cut0v[s] = jnp.full((_L,), cut0s, jnp.int32)
        cut1v[s] = jnp.full((_L,), cut1s, jnp.int32)
        actv[s] = jnp.full((_L,), act0s, jnp.int32)
        return carry

    lax.fori_loop(0, _C, init_chunk, 0)

    def layer_a(i, carry):
        def one(c, cr):
            s = pl.ds(pl.multiple_of(c * _L, 8), _L)
            bs = (b0[s], b1[s], b2[s], b3[s])
            c0, c1 = cut0v[s], cut1v[s]
            bc0 = _sel4(c0, *bs)
            bc1 = _sel4(c1, *bs)
            ch = bc0 - bc1 > 0
            aa = jnp.where(ch, c0, c1)
            kk = jnp.where(ch, c1, c0)
            chd = jnp.where(ch, ch1v[s], ch0v[s])
            g = jnp.maximum(chd, 0)
            valid = ((chd != -1) & (cellv[s] != -1)
                     & (jnp.minimum(actv[s], _LAYERS) == i))
            nch0 = plsc.load_gather(vtab, [g, jnp.full((_L,), 0, jnp.int32)])
            nch1 = plsc.load_gather(vtab, [g, jnp.full((_L,), 1, jnp.int32)])
            ncut0 = plsc.load_gather(vtab, [g, jnp.full((_L,), 2, jnp.int32)])
            ncut1 = plsc.load_gather(vtab, [g, jnp.full((_L,), 3, jnp.int32)])
            nact = plsc.load_gather(vtab, [g, jnp.full((_L,), 4, jnp.int32)])
            vid = plsc.load_gather(vtab, [g, 5 + aa])
            ba = _sel4(aa, *bs)
            for r in range(4):
                isa = aa == r
                isk = kk == r
                nbr = jnp.where(isa, 2.0 * ba, jnp.where(isk, bs[r] - ba, bs[r]))
                brefs[r][s] = jnp.where(valid, nbr, bs[r])
                pr = prefs[r][s]
                prefs[r][s] = jnp.where(valid & isa, vid, pr)
            cellv[s] = jnp.where(valid, chd, cellv[s])
            ch0v[s] = jnp.where(valid, nch0, ch0v[s])
            ch1v[s] = jnp.where(valid, nch1, ch1v[s])
            cut0v[s] = jnp.where(valid, ncut0, cut0v[s])
            cut1v[s] = jnp.where(valid, ncut1, cut1v[s])
            actv[s] = jnp.where(valid, nact, actv[s])
            return cr

        lax.fori_loop(0, _C, one, 0)
        return carry

    lax.fori_loop(1, 10, layer_a, 0)

    def layer(i, carry):
        def pre(c, cr):
            s = pl.ds(pl.multiple_of(c * _L, 8), _L)
            bs = (b0[s], b1[s], b2[s], b3[s])
            c0, c1 = cut0v[s], cut1v[s]
            bc0 = _sel4(c0, *bs)
            bc1 = _sel4(c1, *bs)
            ch = bc0 - bc1 > 0
            av[s] = jnp.where(ch, c0, c1)
            kv[s] = jnp.where(ch, c1, c0)
            child = jnp.where(ch, ch1v[s], ch0v[s])
            childv[s] = child
            gidx[s] = jnp.maximum(child, 0)
            return cr

        lax.fori_loop(0, _C, pre, 0)

        def fire(g, cr):
            o = pl.ds(pl.multiple_of(g * _G, 8), _G)
            pltpu.async_copy(packed2.at[gidx.at[o]], npack.at[o], sem)
            return cr

        def drain(g, cr):
            o = pl.ds(pl.multiple_of(g * _G, 8), _G)
            pltpu.make_async_copy(packed2.at[gidx.at[o]], npack.at[o], sem).wait()
            return cr

        lax.fori_loop(0, _NG, fire, 0)
        lax.fori_loop(0, _NG, drain, 0)

        def post(c, cr):
            s = pl.ds(pl.multiple_of(c * _L, 8), _L)
            chd = childv[s]
            valid = ((chd != -1) & (cellv[s] != -1)
                     & (jnp.minimum(actv[s], _LAYERS) == i))
            aa, kk = av[s], kv[s]
            ri = c * _L + lane
            nch0 = plsc.load_gather(npack, [ri, jnp.full((_L,), 0, jnp.int32)])
            nch1 = plsc.load_gather(npack, [ri, jnp.full((_L,), 1, jnp.int32)])
            ncut0 = plsc.load_gather(npack, [ri, jnp.full((_L,), 2, jnp.int32)])
            ncut1 = plsc.load_gather(npack, [ri, jnp.full((_L,), 3, jnp.int32)])
            nact = plsc.load_gather(npack, [ri, jnp.full((_L,), 4, jnp.int32)])
            vid = plsc.load_gather(npack, [ri, 5 + aa])
            bs = (b0[s], b1[s], b2[s], b3[s])
            ba = _sel4(aa, *bs)
            for r in range(4):
                isa = aa == r
                isk = kk == r
                nbr = jnp.where(isa, 2.0 * ba, jnp.where(isk, bs[r] - ba, bs[r]))
                brefs[r][s] = jnp.where(valid, nbr, bs[r])
                pr = prefs[r][s]
                prefs[r][s] = jnp.where(valid & isa, vid, pr)
            cellv[s] = jnp.where(valid, chd, cellv[s])
            ch0v[s] = jnp.where(valid, nch0, ch0v[s])
            ch1v[s] = jnp.where(valid, nch1, ch1v[s])
            cut0v[s] = jnp.where(valid, ncut0, cut0v[s])
            cut1v[s] = jnp.where(valid, ncut1, cut1v[s])
            actv[s] = jnp.where(valid, nact, actv[s])
            return cr

        lax.fori_loop(0, _C, post, 0)
        return carry

    lax.fori_loop(10, _LAYERS - 1, layer, 0)

    rows = (rows0, rows1, rows2, rows3)

    def interp_block(blk, carry):
        off = pl.multiple_of(blk * _GB, 8)
        st0 = off - (off // _Q) * _Q
        for r in range(4):
            for q in range(4):
                oq = pl.ds(pl.multiple_of(off + q * 32, 8), 32)
                dq = pl.ds(pl.multiple_of(q * 32, 8), 32)
                pltpu.async_copy(field.at[prefs[r].at[oq]], rows[r].at[dq], sem)
        for r in range(4):
            for q in range(4):
                oq = pl.ds(pl.multiple_of(off + q * 32, 8), 32)
                dq = pl.ds(pl.multiple_of(q * 32, 8), 32)
                pltpu.make_async_copy(field.at[prefs[r].at[oq]], rows[r].at[dq], sem).wait()

        def sub(j, cr):
            ps = pl.ds(pl.multiple_of(off + j * _L, 8), _L)
            w = [b0[ps], b1[ps], b2[ps], b3[ps]]
            rowidx = j * _L + lane
            stpos = st0 + j * _L + lane
            for f in range(_F):
                col = jnp.full((_L,), f, jnp.int32)
                acc = w[0] * plsc.load_gather(rows0, [rowidx, col])
                acc = acc + w[1] * plsc.load_gather(rows1, [rowidx, col])
                acc = acc + w[2] * plsc.load_gather(rows2, [rowidx, col])
                acc = acc + w[3] * plsc.load_gather(rows3, [rowidx, col])
                plsc.store_scatter(outstage, [stpos, col], acc)
            return cr

        lax.fori_loop(0, _GB // _L, sub, 0)
        return carry

    nblk_part = _Q // _GB
    for part in range(4):
        lax.fori_loop(part * nblk_part, (part + 1) * nblk_part, interp_block, 0)
        pltpu.sync_copy(outstage, out.at[pl.ds(base + part * _Q, _Q)])


_vmem = pltpu.VMEM
_scratch = [
    _vmem((_TV, _W), jnp.int32),
    _vmem((_P * 3,), jnp.float32),
    _vmem((16,), jnp.float32), _vmem((1, _W), jnp.int32),
    _vmem((_P,), jnp.float32), _vmem((_P,), jnp.float32),
    _vmem((_P,), jnp.float32), _vmem((_P,), jnp.float32),
    _vmem((_P,), jnp.int32), _vmem((_P,), jnp.int32),
    _vmem((_P,), jnp.int32), _vmem((_P,), jnp.int32),
    _vmem((_P,), jnp.int32), _vmem((_P,), jnp.int32),
    _vmem((_P,), jnp.int32), _vmem((_P,), jnp.int32),
    _vmem((_P,), jnp.int32), _vmem((_P,), jnp.int32),
    _vmem((_P,), jnp.int32), _vmem((_P,), jnp.int32),
    _vmem((_P,), jnp.int32), _vmem((_P,), jnp.int32),
    _vmem((_P, _W), jnp.int32),
    _vmem((_GB, _F), jnp.float32), _vmem((_GB, _F), jnp.float32),
    _vmem((_GB, _F), jnp.float32), _vmem((_GB, _F), jnp.float32),
    _vmem((_Q, _F), jnp.float32),
    pltpu.SemaphoreType.DMA,
]

_descend = functools.partial(
    pl.kernel,
    out_type=jax.ShapeDtypeStruct((_N, _F), jnp.float32),
    mesh=plsc.VectorSubcoreMesh(core_axis_name="c", subcore_axis_name="s"),
    compiler_params=pltpu.CompilerParams(
        needs_layout_passes=False, use_tc_tiling_on_sc=False),
    scratch_types=_scratch,
)(_body)


def kernel(xyz, field, point_xyz, child_index, point_index, child_cut, activation_layer):
    m = jnp.concatenate(
        [point_xyz[:4], jnp.ones((4, 1), point_xyz.dtype)], axis=1).T
    ainv = jnp.linalg.inv(m).astype(jnp.float32).reshape(16)
    ncell = child_index.shape[0]
    packed = jnp.concatenate(
        [child_index.astype(jnp.int32),
         child_cut.astype(jnp.int32),
         activation_layer.astype(jnp.int32)[:, None],
         point_index.astype(jnp.int32),
         jnp.zeros((ncell, _W - 9), jnp.int32)],
        axis=1)
    return _descend(xyz.reshape(-1), field, ainv, packed)
